# baseline (speedup 1.0000x reference)
import jax
import jax.numpy as jnp
from jax import lax
from jax.experimental import pallas as pl
from jax.experimental.pallas import tpu as pltpu
from jax.experimental.pallas import tpu_sc as plsc

_B, _H, _S, _D = 8, 16, 4096, 128
_Q = 16
_BH = _B * _H
_NC, _NS = 2, 16
_NW = _NC * _NS
_BH_PER_W = _BH // _NW


def _sc_body(pos_hbm, src_hbm, kval_hbm, vval_hbm, kc_hbm, vc_hbm,
             kout_hbm, vout_hbm, idx_v, rows_v, sem_c, sem_s):
    wid = lax.axis_index("s") * _NC + lax.axis_index("c")

    pltpu.sync_copy(pos_hbm, idx_v.at[0])
    pltpu.sync_copy(src_hbm, idx_v.at[1])
    pos_v = idx_v[0]
    src_v = idx_v[1]

    copies = []
    for i in range(_BH_PER_W):
        r0 = (wid * _BH_PER_W + i) * _S
        for c_hbm, o_hbm in ((kc_hbm, kout_hbm), (vc_hbm, vout_hbm)):
            copies.append(
                pltpu.async_copy(c_hbm.at[pl.ds(r0, _S)],
                                 o_hbm.at[pl.ds(r0, _S)], sem_c))
    for c in copies:
        c.wait()

    for i in range(_BH_PER_W):
        bh = wid * _BH_PER_W + i
        gsrc = src_v + bh * _Q
        gdst = pos_v + bh * _S
        for val_hbm, o_hbm in ((kval_hbm, kout_hbm), (vval_hbm, vout_hbm)):
            pltpu.async_copy(val_hbm.at[gsrc], rows_v, sem_s).wait()
            pltpu.async_copy(rows_v, o_hbm.at[gdst], sem_s).wait()


def kernel(input_pos, k_val, v_val, k_cache, v_cache):
    pos = input_pos.astype(jnp.int32)
    src = (jnp.searchsorted(pos, pos, side="right") - 1).astype(jnp.int32)
    kv = k_val.reshape(_BH * _Q, _D)
    vv = v_val.reshape(_BH * _Q, _D)
    kc = k_cache.reshape(_BH * _S, _D)
    vc = v_cache.reshape(_BH * _S, _D)
    mesh = plsc.VectorSubcoreMesh(core_axis_name="c", subcore_axis_name="s",
                                  num_cores=_NC, num_subcores=_NS)
    kfn = pl.kernel(
        _sc_body,
        out_type=(jax.ShapeDtypeStruct((_BH * _S, _D), jnp.float32),
                  jax.ShapeDtypeStruct((_BH * _S, _D), jnp.float32)),
        mesh=mesh,
        scratch_types=[
            pltpu.VMEM((2, _Q), jnp.int32),
            pltpu.VMEM((_Q, _D), jnp.float32),
            pltpu.SemaphoreType.DMA,
            pltpu.SemaphoreType.DMA,
        ],
    )
    k_out, v_out = kfn(pos, src, kv, vv, kc, vc)
    return (k_out.reshape(_B, _H, _S, _D), v_out.reshape(_B, _H, _S, _D))
